# baseline (speedup 1.0000x reference)
import functools

import jax
import jax.numpy as jnp
from jax import lax
from jax.experimental import pallas as pl
from jax.experimental.pallas import tpu as pltpu
from jax.experimental.pallas import tpu_sc as plsc

_G = 128


@functools.cache
def _make_gather(n_rows, d):
    info = plsc.get_sparse_core_info()
    nc, ns = info.num_cores, info.num_subcores
    nw = nc * ns
    chunks = n_rows // _G
    per_w = chunks // nw
    mesh = plsc.VectorSubcoreMesh(core_axis_name="c", subcore_axis_name="s")

    @functools.partial(
        pl.kernel,
        mesh=mesh,
        out_type=jax.ShapeDtypeStruct((n_rows, d), jnp.float32),
        compiler_params=pltpu.CompilerParams(use_tc_tiling_on_sc=False),
        scratch_types=[
            pltpu.VMEM((per_w, _G), jnp.int32),
            pltpu.VMEM((_G, d), jnp.float32),
            pltpu.SemaphoreType.DMA,
        ],
    )
    def k(idx_hbm, table_hbm, out_hbm, idx_v, rows_v, sem):
        wid = lax.axis_index("s") * nc + lax.axis_index("c")
        cbase = wid * per_w
        pltpu.sync_copy(idx_hbm.at[pl.ds(cbase, per_w)], idx_v)

        def body(j, carry):
            pltpu.async_copy(table_hbm.at[idx_v.at[j]], rows_v, sem).wait()
            pltpu.sync_copy(rows_v, out_hbm.at[pl.ds((cbase + j) * _G, _G)])
            return carry

        lax.fori_loop(0, per_w, body, 0)

    return k


def kernel(x, table):
    b, h = x.shape
    _, d = table.shape
    n = b * h
    idx = x.reshape(n // _G, _G).astype(jnp.int32)
    out = _make_gather(n, d)(idx, table)
    return out.reshape(b, h, d)


# speedup vs baseline: 1.1108x; 1.1108x over previous
import functools

import jax
import jax.numpy as jnp
from jax import lax
from jax.experimental import pallas as pl
from jax.experimental.pallas import tpu as pltpu
from jax.experimental.pallas import tpu_sc as plsc

_G = 128
_NB = 8
_LA = 4


@functools.cache
def _make_gather(n_rows, d):
    info = plsc.get_sparse_core_info()
    nc, ns = info.num_cores, info.num_subcores
    nw = nc * ns
    chunks = n_rows // _G
    per_w = chunks // nw
    groups = (per_w - _LA) // _NB
    assert per_w == _LA + groups * _NB + _LA
    mesh = plsc.VectorSubcoreMesh(core_axis_name="c", subcore_axis_name="s")

    @functools.partial(
        pl.kernel,
        mesh=mesh,
        out_type=jax.ShapeDtypeStruct((n_rows, d), jnp.float32),
        compiler_params=pltpu.CompilerParams(use_tc_tiling_on_sc=False),
        scratch_types=[
            pltpu.VMEM((per_w, _G), jnp.int32),
            pltpu.VMEM((_NB, _G, d), jnp.float32),
        ]
        + [pltpu.SemaphoreType.DMA] * (2 * _NB),
    )
    def k(idx_hbm, table_hbm, out_hbm, idx_v, rows_v, *sems):
        gsem, ssem = sems[:_NB], sems[_NB:]
        wid = lax.axis_index("s") * nc + lax.axis_index("c")
        cbase = wid * per_w
        pltpu.sync_copy(idx_hbm.at[pl.ds(cbase, per_w)], idx_v)

        def fire_gather(c, b):
            pltpu.async_copy(table_hbm.at[idx_v.at[c]], rows_v.at[b], gsem[b])

        def fire_store(c, b):
            pltpu.async_copy(
                rows_v.at[b], out_hbm.at[pl.ds((cbase + c) * _G, _G)], ssem[b]
            )

        def wait_gather(c, b):
            pltpu.make_async_copy(table_hbm.at[idx_v.at[c]], rows_v.at[b], gsem[b]).wait()

        def wait_store(c, b):
            pltpu.make_async_copy(
                rows_v.at[b], out_hbm.at[pl.ds((cbase + c) * _G, _G)], ssem[b]
            ).wait()

        for c in range(_LA):
            fire_gather(c, c % _NB)
        for c in range(_LA):
            wait_gather(c, c % _NB)
            fire_store(c, c % _NB)
            fire_gather(c + _LA, (c + _LA) % _NB)

        def body(g, carry):
            c0 = _LA + g * _NB
            for u in range(_NB):
                c = c0 + u
                b = (_LA + u) % _NB
                bb = u
                wait_gather(c, b)
                fire_store(c, b)
                wait_store(c - _LA, bb)
                fire_gather(c + _LA, bb)
            return carry

        lax.fori_loop(0, groups, body, 0)

        tail = _LA + groups * _NB
        for c in range(tail, per_w):
            b = c % _NB
            wait_gather(c, b)
            fire_store(c, b)
        for c in range(per_w - _NB, per_w):
            wait_store(c, c % _NB)

    return k


def kernel(x, table):
    b, h = x.shape
    _, d = table.shape
    n = b * h
    idx = x.reshape(n // _G, _G).astype(jnp.int32)
    out = _make_gather(n, d)(idx, table)
    return out.reshape(b, h, d)
